# Initial kernel scaffold; baseline (speedup 1.0000x reference)
import jax
import jax.numpy as jnp
from jax.experimental import pallas as pl


def kernel(x, edge_index, W, att_src, att_dst, bias, gamma, beta):
    raise NotImplementedError("write your pallas kernel here")


# speedup vs baseline: 23.2995x; 23.2995x over previous
import functools

import jax
import jax.numpy as jnp
from jax import lax
from jax.experimental import pallas as pl
from jax.experimental.pallas import tpu as pltpu
from jax.experimental.pallas import tpu_sc as plsc

_N = 10000
_E = 320000
_D = 128
_E2 = _E + _N
_NW = 32
_CH = 128
_NCH = -(-_E2 // (_NW * _CH))
_EPW = _NCH * _CH
_EP = _EPW * _NW
_NP = 10240
_RPS = _NP // 16


def _prep_body(x_ref, w_ref, as_ref, ad_ref, h_ref, av_ref, dv_ref, m_ref):
    h = jnp.dot(x_ref[...], w_ref[...], preferred_element_type=jnp.float32)
    h_ref[...] = h
    a_s = jnp.sum(h * as_ref[...], axis=1, keepdims=True)
    a_d = jnp.sum(h * ad_ref[...], axis=1, keepdims=True)
    av_ref[...] = a_s
    dv_ref[...] = a_d
    mm = jnp.max(a_s) + jnp.max(a_d)
    mm = jnp.where(mm >= 0, mm, 0.2 * mm)
    m_ref[...] = jnp.full((1, 1), mm, dtype=jnp.float32)


_prep = pl.pallas_call(
    _prep_body,
    out_shape=[
        jax.ShapeDtypeStruct((_N, _D), jnp.float32),
        jax.ShapeDtypeStruct((_N, 1), jnp.float32),
        jax.ShapeDtypeStruct((_N, 1), jnp.float32),
        jax.ShapeDtypeStruct((1, 1), jnp.float32),
    ],
)


def _sc_body(h_hbm, asrc_hbm, adst_hbm, src_hbm, dst_hbm, m_hbm,
             z_hbm, zs_hbm,
             acc_out, s_out,
             asrc_l, adst_l, src_c, dst_c, w_buf, rows, m_l,
             acc_sh, s_sh, sem):
    c = lax.axis_index("c")
    t = lax.axis_index("s")
    wid = c * 16 + t

    pltpu.sync_copy(z_hbm.at[pl.ds(t * _RPS, _RPS)],
                    acc_sh.at[pl.ds(t * _RPS, _RPS)])
    pltpu.sync_copy(zs_hbm.at[pl.ds(t * _RPS, _RPS)],
                    s_sh.at[pl.ds(t * _RPS, _RPS)])
    pltpu.sync_copy(asrc_hbm, asrc_l)
    pltpu.sync_copy(adst_hbm, adst_l)
    pltpu.sync_copy(m_hbm, m_l)
    plsc.subcore_barrier()

    mvec = m_l[...]

    def chunk(tc, carry):
        base = wid * _EPW + tc * _CH
        pltpu.sync_copy(src_hbm.at[pl.ds(base, _CH)], src_c)
        pltpu.sync_copy(dst_hbm.at[pl.ds(base, _CH)], dst_c)
        cp = pltpu.async_copy(h_hbm.at[src_c], rows, sem)
        for g in range(8):
            s16 = src_c[pl.ds(g * 16, 16)]
            d16 = dst_c[pl.ds(g * 16, 16)]
            a_s = plsc.load_gather(asrc_l, [s16])
            a_d = plsc.load_gather(adst_l, [d16])
            z = a_s + a_d
            e = jnp.where(z >= 0, z, 0.2 * z)
            w = jnp.exp(e - mvec)
            eid = base + g * 16 + lax.iota(jnp.int32, 16)
            w = jnp.where(eid < _E2, w, 0.0)
            w_buf[pl.ds(g * 16, 16)] = w
        cp.wait()

        def scale(j, _):
            jv = jnp.full((16,), j, dtype=jnp.int32)
            wv = plsc.load_gather(w_buf, [jv])
            for k in range(8):
                rows[j, pl.ds(k * 16, 16)] = rows[j, pl.ds(k * 16, 16)] * wv
            return 0

        lax.fori_loop(0, _CH, scale, 0, unroll=2)
        pltpu.sync_copy(rows, acc_sh.at[dst_c], add=True)
        pltpu.sync_copy(w_buf, s_sh.at[dst_c], add=True)
        return carry

    lax.fori_loop(0, _NCH, chunk, 0)
    plsc.subcore_barrier()
    pltpu.sync_copy(acc_sh.at[pl.ds(t * _RPS, _RPS)],
                    acc_out.at[c, pl.ds(t * _RPS, _RPS)])
    pltpu.sync_copy(s_sh.at[pl.ds(t * _RPS, _RPS)],
                    s_out.at[c, pl.ds(t * _RPS, _RPS)])


_sc_edge = functools.partial(
    pl.kernel,
    out_type=[
        jax.ShapeDtypeStruct((2, _NP, _D), jnp.float32),
        jax.ShapeDtypeStruct((2, _NP), jnp.float32),
    ],
    mesh=plsc.VectorSubcoreMesh(core_axis_name="c", subcore_axis_name="s"),
    compiler_params=pltpu.CompilerParams(
        use_tc_tiling_on_sc=False, needs_layout_passes=False),
    scratch_types=[
        pltpu.VMEM((_N,), jnp.float32),
        pltpu.VMEM((_N,), jnp.float32),
        pltpu.VMEM((_CH,), jnp.int32),
        pltpu.VMEM((_CH,), jnp.int32),
        pltpu.VMEM((_CH,), jnp.float32),
        pltpu.VMEM((_CH, _D), jnp.float32),
        pltpu.VMEM((16,), jnp.float32),
        pltpu.VMEM_SHARED((_NP, _D), jnp.float32),
        pltpu.VMEM_SHARED((_NP,), jnp.float32),
        pltpu.SemaphoreType.DMA,
    ],
)(_sc_body)


def _post_body(acc_ref, s_ref, b_ref, g_ref, be_ref, out_ref):
    a = acc_ref[0, :_N] + acc_ref[1, :_N]
    sv = s_ref[0, :_N] + s_ref[1, :_N]
    o = a / (sv + 1e-16) + b_ref[...]
    mean = jnp.mean(o, axis=0, keepdims=True)
    var = jnp.mean((o - mean) ** 2, axis=0, keepdims=True)
    o = (o - mean) / jnp.sqrt(var + 1e-5) * g_ref[...] + be_ref[...]
    out_ref[...] = jnp.maximum(o, 0.0)


_post = pl.pallas_call(
    _post_body,
    out_shape=jax.ShapeDtypeStruct((_N, _D), jnp.float32),
)


def kernel(x, edge_index, W, att_src, att_dst, bias, gamma, beta):
    h, av, dv, m = _prep(x, W, att_src, att_dst)
    pad = _EP - _E2
    loop = jnp.arange(_N, dtype=jnp.int32)
    zpad = jnp.zeros((pad,), dtype=jnp.int32)
    src = jnp.concatenate([edge_index[0], loop, zpad])
    dst = jnp.concatenate([edge_index[1], loop, zpad])
    mvec = jnp.broadcast_to(m.reshape(1), (16,))
    zeros = jnp.zeros((_NP, _D), dtype=jnp.float32)
    zeros_s = jnp.zeros((_NP,), dtype=jnp.float32)
    acc, s = _sc_edge(h, av.reshape(-1), dv.reshape(-1), src, dst, mvec,
                      zeros, zeros_s)
    out = _post(acc, s.reshape(2, _NP, 1), bias.reshape(1, _D),
                gamma.reshape(1, _D), beta.reshape(1, _D))
    return out
